# baseline (speedup 1.0000x reference)
import functools

import jax
import jax.numpy as jnp
from jax import lax
from jax.experimental import pallas as pl
from jax.experimental.pallas import tpu as pltpu
from jax.experimental.pallas import tpu_sc as plsc

_K = 512
_D = 64
_NB = 8
_NW = 32


def _tc_body(z_ref, em2_ref, iota_ref, idx_ref, norm_ref):
    @pl.when(pl.program_id(0) == 0)
    def _():
        em2 = em2_ref[...]
        norm_ref[...] = jnp.sum(em2 * em2, axis=1, keepdims=True) * 0.25

    for b in range(_NB):
        z = z_ref[b]
        s = jax.lax.dot_general(
            em2_ref[...], z, (((1,), (0,)), ((), ())),
            preferred_element_type=jnp.float32)
        a = s + norm_ref[...]
        m = jnp.min(a, axis=0, keepdims=True)
        oh = (a <= m).astype(jnp.float32)
        fi = jax.lax.dot_general(
            iota_ref[...], oh, (((1,), (0,)), ((), ())),
            preferred_element_type=jnp.float32)
        P = z_ref.shape[-1]
        idx_ref[pl.ds(b * P, P)] = jnp.minimum(
            fi[0] * 128.0 + fi[1], float(_K - 1)).astype(jnp.int32)


def _tc_indices(z_r, e):
    B, C, P = z_r.shape
    k = jnp.arange(_K, dtype=jnp.float32)
    hi = jnp.floor(k / 128.0)
    iota_rows = jnp.concatenate(
        [hi[None, :], (k - hi * 128.0)[None, :],
         jnp.zeros((6, _K), jnp.float32)], axis=0)
    return pl.pallas_call(
        _tc_body,
        grid=(B // _NB,),
        in_specs=[
            pl.BlockSpec((_NB, C, P), lambda i: (i, 0, 0)),
            pl.BlockSpec((_K, _D), lambda i: (0, 0)),
            pl.BlockSpec((8, _K), lambda i: (0, 0)),
        ],
        out_specs=pl.BlockSpec((_NB * P,), lambda i: (i,)),
        out_shape=jax.ShapeDtypeStruct((B * P,), jnp.int32),
        scratch_shapes=[pltpu.VMEM((_K, 1), jnp.float32)],
    )(z_r, e * -2.0, iota_rows)


def _sc_gather(eT, idx, B, P):
    bpw = B // _NW
    groups = P // 16

    mesh = plsc.VectorSubcoreMesh(core_axis_name="c", subcore_axis_name="s")

    @functools.partial(
        pl.kernel,
        mesh=mesh,
        compiler_params=pltpu.CompilerParams(needs_layout_passes=False),
        out_type=jax.ShapeDtypeStruct((B * _D * P,), jnp.float32),
        scratch_types=[
            pltpu.VMEM((_D * _K,), jnp.float32),
            pltpu.VMEM((P,), jnp.int32),
            pltpu.VMEM((_D * P,), jnp.float32),
        ],
    )
    def gather_k(eT_hbm, idx_hbm, out_hbm, tab, idxv, buf):
        wid = lax.axis_index("s") * 2 + lax.axis_index("c")
        pltpu.sync_copy(eT_hbm, tab)

        def batch_body(bloc, _):
            b = wid * bpw + bloc
            pltpu.sync_copy(idx_hbm.at[pl.ds(b * P, P)], idxv)

            def ch_body(c, _):
                base = c * _K

                def g_body(g, _):
                    ids = idxv[pl.ds(g * 16, 16)] + base
                    vals = plsc.load_gather(tab, [ids])
                    buf[pl.ds((c * groups + g) * 16, 16)] = vals
                    return 0

                return lax.fori_loop(0, groups, g_body, 0, unroll=4)

            lax.fori_loop(0, _D, ch_body, 0)
            pltpu.sync_copy(buf, out_hbm.at[pl.ds(b * _D * P, _D * P)])
            return 0

        lax.fori_loop(0, bpw, batch_body, 0)

    return gather_k(eT, idx)


def kernel(z_e, e):
    B, C, H, W = z_e.shape
    P = H * W
    z_r = z_e.reshape(B, C, P)
    idx = _tc_indices(z_r, e)
    out = _sc_gather(e.T.reshape(-1), idx, B, P)
    return out.reshape(B, C, H, W)


# speedup vs baseline: 1.2890x; 1.2890x over previous
import functools

import jax
import jax.numpy as jnp
from jax import lax
from jax.experimental import pallas as pl
from jax.experimental.pallas import tpu as pltpu
from jax.experimental.pallas import tpu_sc as plsc

_K = 512
_D = 64
_NB = 8
_NW = 32


def _tc_body(z_ref, em2_ref, iota_ref, idx_ref, norm_ref):
    @pl.when(pl.program_id(0) == 0)
    def _():
        em2 = em2_ref[...]
        norm_ref[...] = jnp.sum(em2 * em2, axis=1, keepdims=True) * 0.25

    for b in range(_NB):
        z = z_ref[b]
        s = jax.lax.dot_general(
            em2_ref[...], z, (((1,), (0,)), ((), ())),
            preferred_element_type=jnp.float32)
        a = s + norm_ref[...]
        m = jnp.min(a, axis=0, keepdims=True)
        oh = (a <= m).astype(jnp.float32)
        fi = jax.lax.dot_general(
            iota_ref[...], oh, (((1,), (0,)), ((), ())),
            preferred_element_type=jnp.float32)
        P = z_ref.shape[-1]
        idx_ref[pl.ds(b * P, P)] = jnp.minimum(
            fi[0] * 128.0 + fi[1], float(_K - 1)).astype(jnp.int32)


def _tc_indices(z_r, e):
    B, C, P = z_r.shape
    k = jnp.arange(_K, dtype=jnp.float32)
    hi = jnp.floor(k / 128.0)
    iota_rows = jnp.concatenate(
        [hi[None, :], (k - hi * 128.0)[None, :],
         jnp.zeros((6, _K), jnp.float32)], axis=0)
    return pl.pallas_call(
        _tc_body,
        grid=(B // _NB,),
        in_specs=[
            pl.BlockSpec((_NB, C, P), lambda i: (i, 0, 0)),
            pl.BlockSpec((_K, _D), lambda i: (0, 0)),
            pl.BlockSpec((8, _K), lambda i: (0, 0)),
        ],
        out_specs=pl.BlockSpec((_NB * P,), lambda i: (i,)),
        out_shape=jax.ShapeDtypeStruct((B * P,), jnp.int32),
        scratch_shapes=[pltpu.VMEM((_K, 1), jnp.float32)],
    )(z_r, e * -2.0, iota_rows)


def _sc_gather(eT, idx, B, P):
    bpw = B // _NW
    groups = P // 16

    mesh = plsc.VectorSubcoreMesh(core_axis_name="c", subcore_axis_name="s")

    @functools.partial(
        pl.kernel,
        mesh=mesh,
        compiler_params=pltpu.CompilerParams(needs_layout_passes=False),
        out_type=jax.ShapeDtypeStruct((B * _D * P,), jnp.float32),
        scratch_types=[
            pltpu.VMEM((_D * _K,), jnp.float32),
            pltpu.VMEM((P,), jnp.int32),
            pltpu.VMEM((_D * P,), jnp.float32),
        ],
    )
    def gather_k(eT_hbm, idx_hbm, out_hbm, tab, idxv, buf):
        wid = lax.axis_index("s") * 2 + lax.axis_index("c")
        pltpu.sync_copy(eT_hbm, tab)

        def batch_body(bloc, _):
            b = wid * bpw + bloc
            pltpu.sync_copy(idx_hbm.at[pl.ds(b * P, P)], idxv)

            def g_body(g, _):
                ids = idxv[pl.ds(g * 16, 16)]
                for c in range(_D):
                    vals = plsc.load_gather(tab, [ids + (c * _K)])
                    buf[pl.ds(c * P + g * 16, 16)] = vals
                return 0

            lax.fori_loop(0, groups, g_body, 0)
            pltpu.sync_copy(buf, out_hbm.at[pl.ds(b * _D * P, _D * P)])
            return 0

        lax.fori_loop(0, bpw, batch_body, 0)

    return gather_k(eT, idx)


def kernel(z_e, e):
    B, C, H, W = z_e.shape
    P = H * W
    z_r = z_e.reshape(B, C, P)
    idx = _tc_indices(z_r, e)
    out = _sc_gather(e.T.reshape(-1), idx, B, P)
    return out.reshape(B, C, H, W)


# speedup vs baseline: 1.4464x; 1.1221x over previous
import functools

import jax
import jax.numpy as jnp
from jax import lax
from jax.experimental import pallas as pl
from jax.experimental.pallas import tpu as pltpu
from jax.experimental.pallas import tpu_sc as plsc

_K = 512
_D = 64
_NB = 8
_NW = 32


def _tc_body(z_ref, em2_ref, iota_ref, idx_ref, norm_ref):
    @pl.when(pl.program_id(0) == 0)
    def _():
        em2 = em2_ref[...]
        norm_ref[...] = jnp.sum(em2 * em2, axis=1, keepdims=True) * 0.25

    for b in range(_NB):
        z = z_ref[b]
        s = jax.lax.dot_general(
            em2_ref[...], z, (((1,), (0,)), ((), ())),
            preferred_element_type=jnp.float32)
        a = s + norm_ref[...]
        m = jnp.min(a, axis=0, keepdims=True)
        oh = (a <= m).astype(jnp.float32)
        fi = jax.lax.dot_general(
            iota_ref[...], oh, (((1,), (0,)), ((), ())),
            preferred_element_type=jnp.float32)
        P = z_ref.shape[-1]
        idx_ref[pl.ds(b * P, P)] = jnp.minimum(
            fi[0] * 128.0 + fi[1], float(_K - 1)).astype(jnp.int32)


def _tc_indices(z_r, e):
    B, C, P = z_r.shape
    k = jnp.arange(_K, dtype=jnp.float32)
    hi = jnp.floor(k / 128.0)
    iota_rows = jnp.concatenate(
        [hi[None, :], (k - hi * 128.0)[None, :],
         jnp.zeros((6, _K), jnp.float32)], axis=0)
    return pl.pallas_call(
        _tc_body,
        grid=(B // _NB,),
        in_specs=[
            pl.BlockSpec((_NB, C, P), lambda i: (i, 0, 0)),
            pl.BlockSpec((_K, _D), lambda i: (0, 0)),
            pl.BlockSpec((8, _K), lambda i: (0, 0)),
        ],
        out_specs=pl.BlockSpec((_NB * P,), lambda i: (i,)),
        out_shape=jax.ShapeDtypeStruct((B * P,), jnp.int32),
        scratch_shapes=[pltpu.VMEM((_K, 1), jnp.float32)],
    )(z_r, e * -2.0, iota_rows)


def _sc_gather(eT, idx, B, P):
    bpw = B // _NW
    groups = P // 16

    mesh = plsc.VectorSubcoreMesh(core_axis_name="c", subcore_axis_name="s")

    @functools.partial(
        pl.kernel,
        mesh=mesh,
        compiler_params=pltpu.CompilerParams(needs_layout_passes=False),
        out_type=jax.ShapeDtypeStruct((B * _D * P,), jnp.float32),
        scratch_types=[
            pltpu.VMEM((_D * _K,), jnp.float32),
            pltpu.VMEM((P,), jnp.int32),
            pltpu.VMEM((_D * P,), jnp.float32),
        ],
    )
    def gather_k(eT_hbm, idx_hbm, out_hbm, tab, idxv, buf):
        wid = lax.axis_index("s") * 2 + lax.axis_index("c")
        pltpu.sync_copy(eT_hbm, tab)

        def batch_body(bloc, _):
            b = wid * bpw + bloc
            pltpu.sync_copy(idx_hbm.at[pl.ds(b * P, P)], idxv)

            @plsc.parallel_loop(0, groups, step=1, unroll=2)
            def g_body(g):
                ids = idxv[pl.ds(g * 16, 16)]
                for c in range(_D):
                    vals = plsc.load_gather(tab, [ids + (c * _K)])
                    buf[pl.ds(c * P + g * 16, 16)] = vals
            pltpu.sync_copy(buf, out_hbm.at[pl.ds(b * _D * P, _D * P)])
            return 0

        lax.fori_loop(0, bpw, batch_body, 0)

    return gather_k(eT, idx)


def kernel(z_e, e):
    B, C, H, W = z_e.shape
    P = H * W
    z_r = z_e.reshape(B, C, P)
    idx = _tc_indices(z_r, e)
    out = _sc_gather(e.T.reshape(-1), idx, B, P)
    return out.reshape(B, C, H, W)


# speedup vs baseline: 3.0057x; 2.0781x over previous
import jax
import jax.numpy as jnp
from jax.experimental import pallas as pl
from jax.experimental.pallas import tpu as pltpu

_K = 512
_D = 64
_NB = 16


def _body(z_ref, em2_ref, eT_ref, out_ref, norm_ref):
    @pl.when(pl.program_id(0) == 0)
    def _():
        em2 = em2_ref[...]
        norm_ref[...] = jnp.sum(em2 * em2, axis=1, keepdims=True) * 0.25

    for b in range(_NB):
        z = z_ref[b]
        s = jax.lax.dot_general(
            em2_ref[...], z, (((1,), (0,)), ((), ())),
            preferred_element_type=jnp.float32)
        a = s + norm_ref[...]
        m = jnp.min(a, axis=0, keepdims=True)
        oh = (a <= m).astype(jnp.float32)
        oa = jax.lax.dot_general(
            eT_ref[...], oh, (((1,), (0,)), ((), ())),
            preferred_element_type=jnp.float32)
        out_ref[b] = oa[:_D] / oa[_D:_D + 1]


def kernel(z_e, e):
    B, C, H, W = z_e.shape
    P = H * W
    z_r = z_e.reshape(B, C, P)
    eT_aug = jnp.concatenate(
        [e.T,
         jnp.ones((1, _K), jnp.float32),
         jnp.zeros((7, _K), jnp.float32)], axis=0)
    out = pl.pallas_call(
        _body,
        grid=(B // _NB,),
        in_specs=[
            pl.BlockSpec((_NB, C, P), lambda i: (i, 0, 0)),
            pl.BlockSpec((_K, _D), lambda i: (0, 0)),
            pl.BlockSpec((_D + 8, _K), lambda i: (0, 0)),
        ],
        out_specs=pl.BlockSpec((_NB, C, P), lambda i: (i, 0, 0)),
        out_shape=jax.ShapeDtypeStruct((B, C, P), jnp.float32),
        scratch_shapes=[pltpu.VMEM((_K, 1), jnp.float32)],
    )(z_r, e * -2.0, eT_aug)
    return out.reshape(B, C, H, W)
